# baseline (speedup 1.0000x reference)
import jax
import jax.numpy as jnp
from jax import lax
from jax.experimental import pallas as pl
from jax.experimental.pallas import tpu as pltpu
from jax.experimental.pallas import tpu_sc as plsc

B = 16384
VOCAB_HALF = 500000
W = 10
K = 20
D = 64
PAD = 128

NC = 2
NS = 16
NW = NC * NS
NB = B // NW
CB = 8
NCH = NB // CB
TP = 16


def _sc_logits_kernel(ctx_sup_hbm, ctx_par_hbm, tgt_sup_hbm, tgt_par_hbm,
                      neg_sup_hbm, neg_par_hbm, itab_hbm, otab_hbm,
                      out_hbm,
                      sup_ctx_a, par_ctx_a, sup_tgt_a, par_tgt_a,
                      sup_neg_a, par_neg_a,
                      ctx_rows_a, tgt_rows_a, neg_rows_a,
                      sup_ctx_b, par_ctx_b, sup_tgt_b, par_tgt_b,
                      sup_neg_b, par_neg_b,
                      ctx_rows_b, tgt_rows_b, neg_rows_b,
                      part_v, logits_v, sem_ga, sem_gb, sem_ia, sem_ib):
  wid = lax.axis_index("s") * NC + lax.axis_index("c")
  lane = lax.iota(jnp.int32, 16)
  bufs = (
      (sup_ctx_a, par_ctx_a, sup_tgt_a, par_tgt_a, sup_neg_a, par_neg_a,
       ctx_rows_a, tgt_rows_a, neg_rows_a, sem_ga, sem_ia),
      (sup_ctx_b, par_ctx_b, sup_tgt_b, par_tgt_b, sup_neg_b, par_neg_b,
       ctx_rows_b, tgt_rows_b, neg_rows_b, sem_gb, sem_ib),
  )

  def i_copies(c, s):
    rb = wid * NB + c * CB
    sc, pc, st, pt, sn, pn = bufs[s][:6]
    si = bufs[s][10]
    return (
        pltpu.make_async_copy(ctx_sup_hbm.at[pl.ds(rb * W, CB * W)], sc, si),
        pltpu.make_async_copy(ctx_par_hbm.at[pl.ds(rb * W, CB * W)], pc, si),
        pltpu.make_async_copy(
            tgt_sup_hbm.at[pl.ds(rb, CB)], st.at[pl.ds(0, CB)], si),
        pltpu.make_async_copy(
            tgt_par_hbm.at[pl.ds(rb, CB)], pt.at[pl.ds(0, CB)], si),
        pltpu.make_async_copy(neg_sup_hbm.at[pl.ds(rb * K, CB * K)], sn, si),
        pltpu.make_async_copy(neg_par_hbm.at[pl.ds(rb * K, CB * K)], pn, si),
    )

  def g_copies(s):
    sc, _, st, _, sn, _, cr, tr, nr, sg, _ = bufs[s]
    return (
        pltpu.make_async_copy(itab_hbm.at[sc], cr, sg),
        pltpu.make_async_copy(otab_hbm.at[st], tr, sg),
        pltpu.make_async_copy(otab_hbm.at[sn], nr, sg),
    )

  def issue(cps):
    for cp in cps:
      cp.start()

  def drain(cps):
    for cp in cps:
      cp.wait()

  def compute(s, c):
    par_ctx_v = bufs[s][1]
    par_tgt_v = bufs[s][3]
    par_neg_v = bufs[s][5]
    ctx_rows_v, tgt_rows_v, neg_rows_v = bufs[s][6], bufs[s][7], bufs[s][8]

    def elem_body(b, carry2):
      def half_cols(par_ref, j):
        po = plsc.load_gather(par_ref, [jnp.full((16,), j, jnp.int32)])
        return po + lane

      acc = [None] * (D // 16)
      for w in range(W):
        j = b * W + w
        row = jnp.full((16,), j, jnp.int32)
        cols = half_cols(par_ctx_v, j)
        for v in range(D // 16):
          g = plsc.load_gather(ctx_rows_v, [row, cols + (v * 16)])
          acc[v] = g if acc[v] is None else acc[v] + g
      mean = [a * jnp.float32(1.0 / W) for a in acc]

      def dot_partial(rows_ref, par_ref, j, k):
        row = jnp.full((16,), j, jnp.int32)
        cols = half_cols(par_ref, j)
        p = mean[0] * plsc.load_gather(rows_ref, [row, cols])
        for v in range(1, D // 16):
          p = p + mean[v] * plsc.load_gather(rows_ref, [row, cols + (v * 16)])
        idx = lane * PAD + k
        plsc.store_scatter(part_v, [idx], p)

      dot_partial(tgt_rows_v, par_tgt_v, b, 0)
      for k in range(K):
        dot_partial(neg_rows_v, par_neg_v, b * K + k, k + 1)
      acc1 = part_v[pl.ds(0, 16)]
      acc2 = part_v[pl.ds(16, 16)]
      for i in range(1, 16):
        acc1 = acc1 + part_v[pl.ds(i * PAD, 16)]
        acc2 = acc2 + part_v[pl.ds(i * PAD + 16, 16)]
      logits_v[b, pl.ds(0, 16)] = acc1
      logits_v[b, pl.ds(16, 16)] = acc2
      return carry2

    lax.fori_loop(0, CB, elem_body, 0)
    rb = wid * NB + c * CB
    pltpu.sync_copy(logits_v, out_hbm.at[pl.ds(rb, CB)])

  zeros16i = jnp.zeros((16,), jnp.int32)
  sup_tgt_a[pl.ds(0, 16)] = zeros16i
  sup_tgt_b[pl.ds(0, 16)] = zeros16i

  issue(i_copies(0, 0))
  drain(i_copies(0, 0))
  issue(g_copies(0))
  issue(i_copies(1, 1))
  T = NCH // 2

  def body(t, carry):
    c0 = 2 * t
    drain(g_copies(0))
    drain(i_copies(c0 + 1, 1))
    issue(g_copies(1))

    @pl.when(t < T - 1)
    def _():
      issue(i_copies(c0 + 2, 0))

    compute(0, c0)
    drain(g_copies(1))

    @pl.when(t < T - 1)
    def _():
      drain(i_copies(c0 + 2, 0))
      issue(g_copies(0))
      issue(i_copies(c0 + 3, 1))

    compute(1, c0 + 1)
    return carry

  lax.fori_loop(0, T, body, 0)


@jax.jit
def _sc_logits(ctx_sup, ctx_par, tgt_sup, tgt_par, neg_sup, neg_par,
               itab2, otab2):
  mesh = plsc.VectorSubcoreMesh(core_axis_name="c", subcore_axis_name="s")
  return pl.kernel(
      _sc_logits_kernel,
      mesh=mesh,
      out_type=jax.ShapeDtypeStruct((B, PAD), jnp.float32),
      compiler_params=pltpu.CompilerParams(
          needs_layout_passes=False, use_tc_tiling_on_sc=True),
      scratch_types=[
          pltpu.VMEM((CB * W,), jnp.int32),
          pltpu.VMEM((CB * W,), jnp.int32),
          pltpu.VMEM((TP,), jnp.int32),
          pltpu.VMEM((TP,), jnp.int32),
          pltpu.VMEM((CB * K,), jnp.int32),
          pltpu.VMEM((CB * K,), jnp.int32),
          pltpu.VMEM((CB * W, 2 * D), jnp.float32),
          pltpu.VMEM((TP, 2 * D), jnp.float32),
          pltpu.VMEM((CB * K, 2 * D), jnp.float32),
          pltpu.VMEM((CB * W,), jnp.int32),
          pltpu.VMEM((CB * W,), jnp.int32),
          pltpu.VMEM((TP,), jnp.int32),
          pltpu.VMEM((TP,), jnp.int32),
          pltpu.VMEM((CB * K,), jnp.int32),
          pltpu.VMEM((CB * K,), jnp.int32),
          pltpu.VMEM((CB * W, 2 * D), jnp.float32),
          pltpu.VMEM((TP, 2 * D), jnp.float32),
          pltpu.VMEM((CB * K, 2 * D), jnp.float32),
          pltpu.VMEM((16 * PAD,), jnp.float32),
          pltpu.VMEM((CB, PAD), jnp.float32),
          pltpu.SemaphoreType.DMA,
          pltpu.SemaphoreType.DMA,
          pltpu.SemaphoreType.DMA,
          pltpu.SemaphoreType.DMA,
      ],
  )(ctx_sup, ctx_par, tgt_sup, tgt_par, neg_sup, neg_par, itab2, otab2)


def _bce_body(l_ref, o_ref):
  x = l_ref[...]
  col = lax.broadcasted_iota(jnp.int32, x.shape, 1)
  label = (col == 0).astype(x.dtype)
  loss = jnp.maximum(x, 0.0) - x * label + jnp.log1p(jnp.exp(-jnp.abs(x)))
  loss = jnp.where(col < (K + 1), loss, 0.0)
  o_ref[0, 0] = jnp.sum(loss) / jnp.float32(B * (K + 1))


@jax.jit
def _bce_mean(logits):
  out = pl.pallas_call(
      _bce_body,
      out_shape=jax.ShapeDtypeStruct((1, 1), jnp.float32),
      in_specs=[pl.BlockSpec(memory_space=pltpu.VMEM)],
      out_specs=pl.BlockSpec(memory_space=pltpu.SMEM),
  )(logits)
  return out[0, 0]


def kernel(context, target, negatives, input_table, output_table):
  ctx = context.astype(jnp.int32).reshape(-1)
  tgt = target.astype(jnp.int32).reshape(-1)
  neg = negatives.astype(jnp.int32).reshape(-1)
  itab2 = input_table.reshape(VOCAB_HALF, 2 * D)
  otab2 = output_table.reshape(VOCAB_HALF, 2 * D)
  logits = _sc_logits(ctx >> 1, (ctx & 1) * D, tgt >> 1, (tgt & 1) * D,
                      neg >> 1, (neg & 1) * D, itab2, otab2)
  return _bce_mean(logits)


# speedup vs baseline: 1.1928x; 1.1928x over previous
import jax
import jax.numpy as jnp
from jax import lax
from jax.experimental import pallas as pl
from jax.experimental.pallas import tpu as pltpu
from jax.experimental.pallas import tpu_sc as plsc

B = 16384
VOCAB = 1000000
W = 10
K = 20
D = 64
PAD = 128

NC = 2
NS = 16
NW = NC * NS
NB = B // NW
CB = 8
NCH = NB // CB

PBLK = 8000


def _pad_body(t_ref, o_ref):
  x = t_ref[...]
  o_ref[:, 0:D] = x
  o_ref[:, D:2 * D] = x


@jax.jit
def _pad128(table):
  return pl.pallas_call(
      _pad_body,
      grid=(VOCAB // PBLK,),
      in_specs=[pl.BlockSpec((PBLK, D), lambda i: (i, 0))],
      out_specs=pl.BlockSpec((PBLK, 2 * D), lambda i: (i, 0)),
      out_shape=jax.ShapeDtypeStruct((VOCAB, 2 * D), jnp.float32),
  )(table)


def _sc_logits_kernel(ctx_hbm, tgt_hbm, neg_hbm, itab_hbm, otab_hbm,
                      out_hbm,
                      ctx_idx_a, tgt_idx_a, neg_idx_a,
                      ctx_rows_a, tgt_rows_a, neg_rows_a,
                      ctx_idx_b, tgt_idx_b, neg_idx_b,
                      ctx_rows_b, tgt_rows_b, neg_rows_b,
                      part_v, logits_v, sem_ga, sem_gb, sem_ia, sem_ib):
  wid = lax.axis_index("s") * NC + lax.axis_index("c")
  lane = lax.iota(jnp.int32, 16)
  bufs = (
      (ctx_idx_a, tgt_idx_a, neg_idx_a, ctx_rows_a, tgt_rows_a, neg_rows_a,
       sem_ga, sem_ia),
      (ctx_idx_b, tgt_idx_b, neg_idx_b, ctx_rows_b, tgt_rows_b, neg_rows_b,
       sem_gb, sem_ib),
  )

  def i_copies(c, s):
    rb = wid * NB + c * CB
    ci, ti, ni = bufs[s][0], bufs[s][1], bufs[s][2]
    si = bufs[s][7]
    return (
        pltpu.make_async_copy(ctx_hbm.at[pl.ds(rb * W, CB * W)], ci, si),
        pltpu.make_async_copy(tgt_hbm.at[pl.ds(rb, CB)], ti, si),
        pltpu.make_async_copy(neg_hbm.at[pl.ds(rb * K, CB * K)], ni, si),
    )

  def g_copies(s):
    ci, ti, ni, cr, tr, nr, sg = bufs[s][:7]
    return (
        pltpu.make_async_copy(itab_hbm.at[ci], cr, sg),
        pltpu.make_async_copy(otab_hbm.at[ti], tr, sg),
        pltpu.make_async_copy(otab_hbm.at[ni], nr, sg),
    )

  def issue(cps):
    for cp in cps:
      cp.start()

  def drain(cps):
    for cp in cps:
      cp.wait()

  def compute(s, c):
    ctx_rows_v, tgt_rows_v, neg_rows_v = bufs[s][3], bufs[s][4], bufs[s][5]

    def elem_body(b, carry2):
      mean = []
      for v in range(D // 16):
        acc = ctx_rows_v[b * W, pl.ds(v * 16, 16)]
        for w in range(1, W):
          acc = acc + ctx_rows_v[b * W + w, pl.ds(v * 16, 16)]
        mean.append(acc * jnp.float32(1.0 / W))

      def dot_partial(rows_ref, r, k):
        p = mean[0] * rows_ref[r, pl.ds(0, 16)]
        for v in range(1, D // 16):
          p = p + mean[v] * rows_ref[r, pl.ds(v * 16, 16)]
        idx = lane * PAD + k
        plsc.store_scatter(part_v, [idx], p)

      dot_partial(tgt_rows_v, b, 0)
      for k in range(K):
        dot_partial(neg_rows_v, b * K + k, k + 1)
      acc1 = part_v[pl.ds(0, 16)]
      acc2 = part_v[pl.ds(16, 16)]
      for i in range(1, 16):
        acc1 = acc1 + part_v[pl.ds(i * PAD, 16)]
        acc2 = acc2 + part_v[pl.ds(i * PAD + 16, 16)]
      logits_v[b, pl.ds(0, 16)] = acc1
      logits_v[b, pl.ds(16, 16)] = acc2
      return carry2

    lax.fori_loop(0, CB, elem_body, 0)
    rb = wid * NB + c * CB
    pltpu.sync_copy(logits_v, out_hbm.at[pl.ds(rb, CB)])

  issue(i_copies(0, 0))
  drain(i_copies(0, 0))
  issue(g_copies(0))
  issue(i_copies(1, 1))
  T = NCH // 2

  def body(t, carry):
    c0 = 2 * t
    drain(g_copies(0))
    drain(i_copies(c0 + 1, 1))
    issue(g_copies(1))

    @pl.when(t < T - 1)
    def _():
      issue(i_copies(c0 + 2, 0))

    compute(0, c0)
    drain(g_copies(1))

    @pl.when(t < T - 1)
    def _():
      drain(i_copies(c0 + 2, 0))
      issue(g_copies(0))
      issue(i_copies(c0 + 3, 1))

    compute(1, c0 + 1)
    return carry

  lax.fori_loop(0, T, body, 0)


@jax.jit
def _sc_logits(ctx_flat, tgt_flat, neg_flat, itab_pad, otab_pad):
  mesh = plsc.VectorSubcoreMesh(core_axis_name="c", subcore_axis_name="s")
  return pl.kernel(
      _sc_logits_kernel,
      mesh=mesh,
      out_type=jax.ShapeDtypeStruct((B, PAD), jnp.float32),
      compiler_params=pltpu.CompilerParams(
          needs_layout_passes=False, use_tc_tiling_on_sc=True),
      scratch_types=[
          pltpu.VMEM((CB * W,), jnp.int32),
          pltpu.VMEM((CB,), jnp.int32),
          pltpu.VMEM((CB * K,), jnp.int32),
          pltpu.VMEM((CB * W, 2 * D), jnp.float32),
          pltpu.VMEM((CB, 2 * D), jnp.float32),
          pltpu.VMEM((CB * K, 2 * D), jnp.float32),
          pltpu.VMEM((CB * W,), jnp.int32),
          pltpu.VMEM((CB,), jnp.int32),
          pltpu.VMEM((CB * K,), jnp.int32),
          pltpu.VMEM((CB * W, 2 * D), jnp.float32),
          pltpu.VMEM((CB, 2 * D), jnp.float32),
          pltpu.VMEM((CB * K, 2 * D), jnp.float32),
          pltpu.VMEM((16 * PAD,), jnp.float32),
          pltpu.VMEM((CB, PAD), jnp.float32),
          pltpu.SemaphoreType.DMA,
          pltpu.SemaphoreType.DMA,
          pltpu.SemaphoreType.DMA,
          pltpu.SemaphoreType.DMA,
      ],
  )(ctx_flat, tgt_flat, neg_flat, itab_pad, otab_pad)


def _bce_body(l_ref, o_ref):
  x = l_ref[...]
  col = lax.broadcasted_iota(jnp.int32, x.shape, 1)
  label = (col == 0).astype(x.dtype)
  loss = jnp.maximum(x, 0.0) - x * label + jnp.log1p(jnp.exp(-jnp.abs(x)))
  loss = jnp.where(col < (K + 1), loss, 0.0)
  o_ref[0, 0] = jnp.sum(loss) / jnp.float32(B * (K + 1))


@jax.jit
def _bce_mean(logits):
  out = pl.pallas_call(
      _bce_body,
      out_shape=jax.ShapeDtypeStruct((1, 1), jnp.float32),
      in_specs=[pl.BlockSpec(memory_space=pltpu.VMEM)],
      out_specs=pl.BlockSpec(memory_space=pltpu.SMEM),
  )(logits)
  return out[0, 0]


def kernel(context, target, negatives, input_table, output_table):
  ctx_flat = context.astype(jnp.int32).reshape(-1)
  tgt_flat = target.astype(jnp.int32).reshape(-1)
  neg_flat = negatives.astype(jnp.int32).reshape(-1)
  logits = _sc_logits(ctx_flat, tgt_flat, neg_flat,
                      _pad128(input_table), _pad128(output_table))
  return _bce_mean(logits)


# speedup vs baseline: 1.2064x; 1.0114x over previous
import jax
import jax.numpy as jnp
from jax import lax
from jax.experimental import pallas as pl
from jax.experimental.pallas import tpu as pltpu
from jax.experimental.pallas import tpu_sc as plsc

B = 16384
VOCAB = 1000000
W = 10
K = 20
D = 64
PAD = 128

NC = 2
NS = 16
NW = NC * NS
NB = B // NW
CB = 8
NCH = NB // CB

PBLK = 8000


def _pad_body(t_ref, o_ref):
  o_ref[:, 0:D] = t_ref[...]


@jax.jit
def _pad128(table):
  return pl.pallas_call(
      _pad_body,
      grid=(VOCAB // PBLK,),
      in_specs=[pl.BlockSpec((PBLK, D), lambda i: (i, 0))],
      out_specs=pl.BlockSpec((PBLK, 2 * D), lambda i: (i, 0)),
      out_shape=jax.ShapeDtypeStruct((VOCAB, 2 * D), jnp.float32),
  )(table)


def _sc_logits_kernel(ctx_hbm, tgt_hbm, neg_hbm, itab_hbm, otab_hbm,
                      out_hbm,
                      ctx_idx_a, tgt_idx_a, neg_idx_a,
                      ctx_rows_a, tgt_rows_a, neg_rows_a,
                      ctx_idx_b, tgt_idx_b, neg_idx_b,
                      ctx_rows_b, tgt_rows_b, neg_rows_b,
                      part_v, logits_v, sem_ga, sem_gb, sem_ia, sem_ib):
  wid = lax.axis_index("s") * NC + lax.axis_index("c")
  lane = lax.iota(jnp.int32, 16)
  bufs = (
      (ctx_idx_a, tgt_idx_a, neg_idx_a, ctx_rows_a, tgt_rows_a, neg_rows_a,
       sem_ga, sem_ia),
      (ctx_idx_b, tgt_idx_b, neg_idx_b, ctx_rows_b, tgt_rows_b, neg_rows_b,
       sem_gb, sem_ib),
  )

  def i_copies(c, s):
    rb = wid * NB + c * CB
    ci, ti, ni = bufs[s][0], bufs[s][1], bufs[s][2]
    si = bufs[s][7]
    return (
        pltpu.make_async_copy(ctx_hbm.at[pl.ds(rb * W, CB * W)], ci, si),
        pltpu.make_async_copy(tgt_hbm.at[pl.ds(rb, CB)], ti, si),
        pltpu.make_async_copy(neg_hbm.at[pl.ds(rb * K, CB * K)], ni, si),
    )

  def g_copies(s):
    ci, ti, ni, cr, tr, nr, sg = bufs[s][:7]
    return (
        pltpu.make_async_copy(itab_hbm.at[ci], cr, sg),
        pltpu.make_async_copy(otab_hbm.at[ti], tr, sg),
        pltpu.make_async_copy(otab_hbm.at[ni], nr, sg),
    )

  def issue(cps):
    for cp in cps:
      cp.start()

  def drain(cps):
    for cp in cps:
      cp.wait()

  def compute(s, c):
    ctx_rows_v, tgt_rows_v, neg_rows_v = bufs[s][3], bufs[s][4], bufs[s][5]

    def elem_body(b, carry2):
      mean = []
      for v in range(D // 16):
        acc = ctx_rows_v[b * W, pl.ds(v * 16, 16)]
        for w in range(1, W):
          acc = acc + ctx_rows_v[b * W + w, pl.ds(v * 16, 16)]
        mean.append(acc * jnp.float32(1.0 / W))

      def dot_partial(rows_ref, r, k):
        p = mean[0] * rows_ref[r, pl.ds(0, 16)]
        for v in range(1, D // 16):
          p = p + mean[v] * rows_ref[r, pl.ds(v * 16, 16)]
        idx = lane * PAD + k
        plsc.store_scatter(part_v, [idx], p)

      dot_partial(tgt_rows_v, b, 0)
      for k in range(K):
        dot_partial(neg_rows_v, b * K + k, k + 1)
      acc1 = part_v[pl.ds(0, 16)]
      acc2 = part_v[pl.ds(16, 16)]
      for i in range(1, 16):
        acc1 = acc1 + part_v[pl.ds(i * PAD, 16)]
        acc2 = acc2 + part_v[pl.ds(i * PAD + 16, 16)]
      logits_v[b, pl.ds(0, 16)] = acc1
      logits_v[b, pl.ds(16, 16)] = acc2
      return carry2

    lax.fori_loop(0, CB, elem_body, 0)
    rb = wid * NB + c * CB
    pltpu.sync_copy(logits_v, out_hbm.at[pl.ds(rb, CB)])

  issue(i_copies(0, 0))
  drain(i_copies(0, 0))
  issue(g_copies(0))
  issue(i_copies(1, 1))
  T = NCH // 2

  def body(t, carry):
    c0 = 2 * t
    drain(g_copies(0))
    drain(i_copies(c0 + 1, 1))
    issue(g_copies(1))

    @pl.when(t < T - 1)
    def _():
      issue(i_copies(c0 + 2, 0))

    compute(0, c0)
    drain(g_copies(1))

    @pl.when(t < T - 1)
    def _():
      drain(i_copies(c0 + 2, 0))
      issue(g_copies(0))
      issue(i_copies(c0 + 3, 1))

    compute(1, c0 + 1)
    return carry

  lax.fori_loop(0, T, body, 0)


@jax.jit
def _sc_logits(ctx_flat, tgt_flat, neg_flat, itab_pad, otab_pad):
  mesh = plsc.VectorSubcoreMesh(core_axis_name="c", subcore_axis_name="s")
  return pl.kernel(
      _sc_logits_kernel,
      mesh=mesh,
      out_type=jax.ShapeDtypeStruct((B, PAD), jnp.float32),
      compiler_params=pltpu.CompilerParams(
          needs_layout_passes=False, use_tc_tiling_on_sc=True),
      scratch_types=[
          pltpu.VMEM((CB * W,), jnp.int32),
          pltpu.VMEM((CB,), jnp.int32),
          pltpu.VMEM((CB * K,), jnp.int32),
          pltpu.VMEM((CB * W, 2 * D), jnp.float32),
          pltpu.VMEM((CB, 2 * D), jnp.float32),
          pltpu.VMEM((CB * K, 2 * D), jnp.float32),
          pltpu.VMEM((CB * W,), jnp.int32),
          pltpu.VMEM((CB,), jnp.int32),
          pltpu.VMEM((CB * K,), jnp.int32),
          pltpu.VMEM((CB * W, 2 * D), jnp.float32),
          pltpu.VMEM((CB, 2 * D), jnp.float32),
          pltpu.VMEM((CB * K, 2 * D), jnp.float32),
          pltpu.VMEM((16 * PAD,), jnp.float32),
          pltpu.VMEM((CB, PAD), jnp.float32),
          pltpu.SemaphoreType.DMA,
          pltpu.SemaphoreType.DMA,
          pltpu.SemaphoreType.DMA,
          pltpu.SemaphoreType.DMA,
      ],
  )(ctx_flat, tgt_flat, neg_flat, itab_pad, otab_pad)


def _bce_body(l_ref, o_ref):
  x = l_ref[...]
  col = lax.broadcasted_iota(jnp.int32, x.shape, 1)
  label = (col == 0).astype(x.dtype)
  loss = jnp.maximum(x, 0.0) - x * label + jnp.log1p(jnp.exp(-jnp.abs(x)))
  loss = jnp.where(col < (K + 1), loss, 0.0)
  o_ref[0, 0] = jnp.sum(loss) / jnp.float32(B * (K + 1))


@jax.jit
def _bce_mean(logits):
  out = pl.pallas_call(
      _bce_body,
      out_shape=jax.ShapeDtypeStruct((1, 1), jnp.float32),
      in_specs=[pl.BlockSpec(memory_space=pltpu.VMEM)],
      out_specs=pl.BlockSpec(memory_space=pltpu.SMEM),
  )(logits)
  return out[0, 0]


def kernel(context, target, negatives, input_table, output_table):
  ctx_flat = context.astype(jnp.int32).reshape(-1)
  tgt_flat = target.astype(jnp.int32).reshape(-1)
  neg_flat = negatives.astype(jnp.int32).reshape(-1)
  logits = _sc_logits(ctx_flat, tgt_flat, neg_flat,
                      _pad128(input_table), _pad128(output_table))
  return _bce_mean(logits)


# speedup vs baseline: 1.3278x; 1.1006x over previous
import jax
import jax.numpy as jnp
from jax import lax
from jax.experimental import pallas as pl
from jax.experimental.pallas import tpu as pltpu
from jax.experimental.pallas import tpu_sc as plsc

B = 16384
W = 10
K = 20
D = 64
PAD = 32

NC = 2
NS = 16
NW = NC * NS
NB = B // NW
CB = 16
NCH = NB // CB


def _sc_logits_kernel(ctx_hbm, tgt_hbm, neg_hbm, itab_hbm, otab_hbm,
                      out_hbm,
                      ctx_idx_a, tgt_idx_a, neg_idx_a,
                      ctx_rows_a, tgt_rows_a, neg_rows_a,
                      ctx_idx_b, tgt_idx_b, neg_idx_b,
                      ctx_rows_b, tgt_rows_b, neg_rows_b,
                      logits_v, sem_ga, sem_gb, sem_ia, sem_ib):
  wid = lax.axis_index("s") * NC + lax.axis_index("c")
  zeros16 = jnp.zeros((16,), jnp.float32)
  bufs = (
      (ctx_idx_a, tgt_idx_a, neg_idx_a, ctx_rows_a, tgt_rows_a, neg_rows_a,
       sem_ga, sem_ia),
      (ctx_idx_b, tgt_idx_b, neg_idx_b, ctx_rows_b, tgt_rows_b, neg_rows_b,
       sem_gb, sem_ib),
  )

  def i_copies(c, s):
    rb = wid * NB + c * CB
    ci, ti, ni = bufs[s][0], bufs[s][1], bufs[s][2]
    si = bufs[s][7]
    return (
        pltpu.make_async_copy(ctx_hbm.at[pl.ds(rb * W, CB * W)], ci, si),
        pltpu.make_async_copy(tgt_hbm.at[pl.ds(rb, CB)], ti, si),
        pltpu.make_async_copy(neg_hbm.at[pl.ds(rb * K, CB * K)], ni, si),
    )

  def g_copies(s):
    ci, ti, ni, cr, tr, nr, sg = bufs[s][:7]
    return (
        pltpu.make_async_copy(itab_hbm.at[ci], cr, sg),
        pltpu.make_async_copy(otab_hbm.at[ti], tr, sg),
        pltpu.make_async_copy(otab_hbm.at[ni], nr, sg),
    )

  def issue(cps):
    for cp in cps:
      cp.start()

  def drain(cps):
    for cp in cps:
      cp.wait()

  def compute(s, c):
    ctx_rows_v, tgt_rows_v, neg_rows_v = bufs[s][3], bufs[s][4], bufs[s][5]

    def elem_body(b, carry2):
      mean = []
      for v in range(D // 16):
        acc = ctx_rows_v[b * W, pl.ds(v * 16, 16)]
        for w in range(1, W):
          acc = acc + ctx_rows_v[b * W + w, pl.ds(v * 16, 16)]
        mean.append(acc)

      logits_v[pl.ds(b * PAD, 16)] = zeros16
      logits_v[pl.ds(b * PAD + 16, 16)] = zeros16

      def dot_add(rows_ref, r, k):
        p = mean[0] * rows_ref[r, pl.ds(0, 16)]
        for v in range(1, D // 16):
          p = p + mean[v] * rows_ref[r, pl.ds(v * 16, 16)]
        idx = jnp.full((16,), b * PAD + k, jnp.int32)
        plsc.addupdate_scatter(logits_v, [idx], p)

      dot_add(tgt_rows_v, b, 0)
      for k in range(K):
        dot_add(neg_rows_v, b * K + k, k + 1)
      return carry2

    lax.fori_loop(0, CB, elem_body, 0)
    rb = wid * NB + c * CB
    pltpu.sync_copy(logits_v, out_hbm.at[pl.ds(rb * PAD, CB * PAD)])

  issue(i_copies(0, 0))
  drain(i_copies(0, 0))
  issue(g_copies(0))
  issue(i_copies(1, 1))
  T = NCH // 2

  def body(t, carry):
    c0 = 2 * t
    drain(g_copies(0))
    drain(i_copies(c0 + 1, 1))
    issue(g_copies(1))

    @pl.when(t < T - 1)
    def _():
      issue(i_copies(c0 + 2, 0))

    compute(0, c0)
    drain(g_copies(1))

    @pl.when(t < T - 1)
    def _():
      drain(i_copies(c0 + 2, 0))
      issue(g_copies(0))
      issue(i_copies(c0 + 3, 1))

    compute(1, c0 + 1)
    return carry

  lax.fori_loop(0, T, body, 0)


@jax.jit
def _sc_logits(ctx_flat, tgt_flat, neg_flat, itab, otab):
  mesh = plsc.VectorSubcoreMesh(core_axis_name="c", subcore_axis_name="s")
  return pl.kernel(
      _sc_logits_kernel,
      mesh=mesh,
      out_type=jax.ShapeDtypeStruct((B * PAD,), jnp.float32),
      compiler_params=pltpu.CompilerParams(
          needs_layout_passes=False, use_tc_tiling_on_sc=False),
      scratch_types=[
          pltpu.VMEM((CB * W,), jnp.int32),
          pltpu.VMEM((CB,), jnp.int32),
          pltpu.VMEM((CB * K,), jnp.int32),
          pltpu.VMEM((CB * W, D), jnp.float32),
          pltpu.VMEM((CB, D), jnp.float32),
          pltpu.VMEM((CB * K, D), jnp.float32),
          pltpu.VMEM((CB * W,), jnp.int32),
          pltpu.VMEM((CB,), jnp.int32),
          pltpu.VMEM((CB * K,), jnp.int32),
          pltpu.VMEM((CB * W, D), jnp.float32),
          pltpu.VMEM((CB, D), jnp.float32),
          pltpu.VMEM((CB * K, D), jnp.float32),
          pltpu.VMEM((CB * PAD,), jnp.float32),
          pltpu.SemaphoreType.DMA,
          pltpu.SemaphoreType.DMA,
          pltpu.SemaphoreType.DMA,
          pltpu.SemaphoreType.DMA,
      ],
  )(ctx_flat, tgt_flat, neg_flat, itab, otab)


def _bce_body(l_ref, o_ref):
  x = l_ref[...] * jnp.float32(1.0 / W)
  col = lax.broadcasted_iota(jnp.int32, x.shape, 1)
  label = (col == 0).astype(x.dtype)
  loss = jnp.maximum(x, 0.0) - x * label + jnp.log1p(jnp.exp(-jnp.abs(x)))
  loss = jnp.where(col < (K + 1), loss, 0.0)
  o_ref[0, 0] = jnp.sum(loss) / jnp.float32(B * (K + 1))


@jax.jit
def _bce_mean(logits):
  out = pl.pallas_call(
      _bce_body,
      out_shape=jax.ShapeDtypeStruct((1, 1), jnp.float32),
      in_specs=[pl.BlockSpec(memory_space=pltpu.VMEM)],
      out_specs=pl.BlockSpec(memory_space=pltpu.SMEM),
  )(logits)
  return out[0, 0]


def kernel(context, target, negatives, input_table, output_table):
  ctx_flat = context.astype(jnp.int32).reshape(-1)
  tgt_flat = target.astype(jnp.int32).reshape(-1)
  neg_flat = negatives.astype(jnp.int32).reshape(-1)
  logits = _sc_logits(ctx_flat, tgt_flat, neg_flat, input_table, output_table)
  return _bce_mean(logits.reshape(B, PAD))


# speedup vs baseline: 1.4240x; 1.0724x over previous
import jax
import jax.numpy as jnp
from jax import lax
from jax.experimental import pallas as pl
from jax.experimental.pallas import tpu as pltpu
from jax.experimental.pallas import tpu_sc as plsc

B = 16384
W = 10
K = 20
D = 64
PAD = 32

NC = 2
NS = 16
NW = NC * NS
NB = B // NW
CB = 16
NCH = NB // CB


def _tree_sum(xs):
  xs = list(xs)
  while len(xs) > 1:
    nxt = [xs[i] + xs[i + 1] for i in range(0, len(xs) - 1, 2)]
    if len(xs) % 2:
      nxt.append(xs[-1])
    xs = nxt
  return xs[0]


def _sc_logits_kernel(ctx_hbm, tgt_hbm, neg_hbm, itab_hbm, otab_hbm,
                      out_hbm,
                      ctx_idx_a, tgt_idx_a, neg_idx_a,
                      ctx_rows_a, tgt_rows_a, neg_rows_a,
                      ctx_idx_b, tgt_idx_b, neg_idx_b,
                      ctx_rows_b, tgt_rows_b, neg_rows_b,
                      part0_v, part1_v, logits_v,
                      sem_ga, sem_gb, sem_ia, sem_ib):
  wid = lax.axis_index("s") * NC + lax.axis_index("c")
  lane = lax.iota(jnp.int32, 16)
  bufs = (
      (ctx_idx_a, tgt_idx_a, neg_idx_a, ctx_rows_a, tgt_rows_a, neg_rows_a,
       sem_ga, sem_ia),
      (ctx_idx_b, tgt_idx_b, neg_idx_b, ctx_rows_b, tgt_rows_b, neg_rows_b,
       sem_gb, sem_ib),
  )

  def i_copies(c, s):
    rb = wid * NB + c * CB
    ci, ti, ni = bufs[s][0], bufs[s][1], bufs[s][2]
    si = bufs[s][7]
    return (
        pltpu.make_async_copy(ctx_hbm.at[pl.ds(rb * W, CB * W)], ci, si),
        pltpu.make_async_copy(tgt_hbm.at[pl.ds(rb, CB)], ti, si),
        pltpu.make_async_copy(neg_hbm.at[pl.ds(rb * K, CB * K)], ni, si),
    )

  def g_copies(s):
    ci, ti, ni, cr, tr, nr, sg = bufs[s][:7]
    return (
        pltpu.make_async_copy(itab_hbm.at[ci], cr, sg),
        pltpu.make_async_copy(otab_hbm.at[ti], tr, sg),
        pltpu.make_async_copy(otab_hbm.at[ni], nr, sg),
    )

  def issue(cps):
    for cp in cps:
      cp.start()

  def drain(cps):
    for cp in cps:
      cp.wait()

  def compute(s, c):
    ctx_rows_v, tgt_rows_v, neg_rows_v = bufs[s][3], bufs[s][4], bufs[s][5]

    def scatter_dots(b, part_v):
      mean = []
      for v in range(D // 16):
        rows = [ctx_rows_v[b * W + w, pl.ds(v * 16, 16)] for w in range(W)]
        mean.append(_tree_sum(rows))

      def dot_partial(rows_ref, r, k):
        prods = [mean[v] * rows_ref[r, pl.ds(v * 16, 16)]
                 for v in range(D // 16)]
        plsc.store_scatter(part_v, [lane * PAD + k], _tree_sum(prods))

      dot_partial(tgt_rows_v, b, 0)
      for k in range(K):
        dot_partial(neg_rows_v, b * K + k, k + 1)

    def reduce_part(b, part_v):
      acc1 = _tree_sum([part_v[pl.ds(i * PAD, 16)] for i in range(16)])
      acc2 = _tree_sum([part_v[pl.ds(i * PAD + 16, 16)] for i in range(16)])
      logits_v[b, pl.ds(0, 16)] = acc1
      logits_v[b, pl.ds(16, 16)] = acc2

    def pair_body(h, carry2):
      b0 = 2 * h
      scatter_dots(b0, part0_v)
      scatter_dots(b0 + 1, part1_v)
      reduce_part(b0, part0_v)
      reduce_part(b0 + 1, part1_v)
      return carry2

    lax.fori_loop(0, CB // 2, pair_body, 0)
    rb = wid * NB + c * CB
    pltpu.sync_copy(logits_v, out_hbm.at[pl.ds(rb, CB)])

  issue(i_copies(0, 0))
  drain(i_copies(0, 0))
  issue(g_copies(0))
  issue(i_copies(1, 1))
  T = NCH // 2

  def body(t, carry):
    c0 = 2 * t
    drain(g_copies(0))
    drain(i_copies(c0 + 1, 1))
    issue(g_copies(1))

    @pl.when(t < T - 1)
    def _():
      issue(i_copies(c0 + 2, 0))

    compute(0, c0)
    drain(g_copies(1))

    @pl.when(t < T - 1)
    def _():
      drain(i_copies(c0 + 2, 0))
      issue(g_copies(0))
      issue(i_copies(c0 + 3, 1))

    compute(1, c0 + 1)
    return carry

  lax.fori_loop(0, T, body, 0)


@jax.jit
def _sc_logits(ctx_flat, tgt_flat, neg_flat, itab, otab):
  mesh = plsc.VectorSubcoreMesh(core_axis_name="c", subcore_axis_name="s")
  return pl.kernel(
      _sc_logits_kernel,
      mesh=mesh,
      out_type=jax.ShapeDtypeStruct((B, PAD), jnp.float32),
      compiler_params=pltpu.CompilerParams(
          needs_layout_passes=False, use_tc_tiling_on_sc=False),
      scratch_types=[
          pltpu.VMEM((CB * W,), jnp.int32),
          pltpu.VMEM((CB,), jnp.int32),
          pltpu.VMEM((CB * K,), jnp.int32),
          pltpu.VMEM((CB * W, D), jnp.float32),
          pltpu.VMEM((CB, D), jnp.float32),
          pltpu.VMEM((CB * K, D), jnp.float32),
          pltpu.VMEM((CB * W,), jnp.int32),
          pltpu.VMEM((CB,), jnp.int32),
          pltpu.VMEM((CB * K,), jnp.int32),
          pltpu.VMEM((CB * W, D), jnp.float32),
          pltpu.VMEM((CB, D), jnp.float32),
          pltpu.VMEM((CB * K, D), jnp.float32),
          pltpu.VMEM((16 * PAD,), jnp.float32),
          pltpu.VMEM((16 * PAD,), jnp.float32),
          pltpu.VMEM((CB, PAD), jnp.float32),
          pltpu.SemaphoreType.DMA,
          pltpu.SemaphoreType.DMA,
          pltpu.SemaphoreType.DMA,
          pltpu.SemaphoreType.DMA,
      ],
  )(ctx_flat, tgt_flat, neg_flat, itab, otab)


def _bce_body(l_ref, o_ref):
  x = l_ref[...] * jnp.float32(1.0 / W)
  col = lax.broadcasted_iota(jnp.int32, x.shape, 1)
  label = (col == 0).astype(x.dtype)
  loss = jnp.maximum(x, 0.0) - x * label + jnp.log1p(jnp.exp(-jnp.abs(x)))
  loss = jnp.where(col < (K + 1), loss, 0.0)
  o_ref[0, 0] = jnp.sum(loss) / jnp.float32(B * (K + 1))


@jax.jit
def _bce_mean(logits):
  out = pl.pallas_call(
      _bce_body,
      out_shape=jax.ShapeDtypeStruct((1, 1), jnp.float32),
      in_specs=[pl.BlockSpec(memory_space=pltpu.VMEM)],
      out_specs=pl.BlockSpec(memory_space=pltpu.SMEM),
  )(logits)
  return out[0, 0]


def kernel(context, target, negatives, input_table, output_table):
  ctx_flat = context.astype(jnp.int32).reshape(-1)
  tgt_flat = target.astype(jnp.int32).reshape(-1)
  neg_flat = negatives.astype(jnp.int32).reshape(-1)
  logits = _sc_logits(ctx_flat, tgt_flat, neg_flat, input_table, output_table)
  return _bce_mean(logits)


# speedup vs baseline: 1.4242x; 1.0001x over previous
import jax
import jax.numpy as jnp
from jax import lax
from jax.experimental import pallas as pl
from jax.experimental.pallas import tpu as pltpu
from jax.experimental.pallas import tpu_sc as plsc

B = 16384
W = 10
K = 20
D = 64
PAD = 32

NC = 2
NS = 16
NW = NC * NS
NB = B // NW
CB = 16
NCH = NB // CB


def _tree_sum(xs):
  xs = list(xs)
  while len(xs) > 1:
    nxt = [xs[i] + xs[i + 1] for i in range(0, len(xs) - 1, 2)]
    if len(xs) % 2:
      nxt.append(xs[-1])
    xs = nxt
  return xs[0]


def _pipeline(i_copies, g_copies, compute):
  def issue(cps):
    for cp in cps:
      cp.start()

  def drain(cps):
    for cp in cps:
      cp.wait()

  issue(i_copies(0, 0))
  drain(i_copies(0, 0))
  issue(g_copies(0))
  issue(i_copies(1, 1))
  T = NCH // 2

  def body(t, carry):
    c0 = 2 * t
    drain(g_copies(0))
    drain(i_copies(c0 + 1, 1))
    issue(g_copies(1))

    @pl.when(t < T - 1)
    def _():
      issue(i_copies(c0 + 2, 0))

    compute(0, c0)
    drain(g_copies(1))

    @pl.when(t < T - 1)
    def _():
      drain(i_copies(c0 + 2, 0))
      issue(g_copies(0))
      issue(i_copies(c0 + 3, 1))

    compute(1, c0 + 1)
    return carry

  lax.fori_loop(0, T, body, 0)


def _sc_sums_kernel(ctx_hbm, itab_hbm, out_hbm,
                    ctx_idx_a, ctx_rows_a, ctx_idx_b, ctx_rows_b,
                    sums_v, sem_ga, sem_gb, sem_ia, sem_ib):
  wid = lax.axis_index("s") * NC + lax.axis_index("c")
  bufs = ((ctx_idx_a, ctx_rows_a, sem_ga, sem_ia),
          (ctx_idx_b, ctx_rows_b, sem_gb, sem_ib))

  def i_copies(c, s):
    rb = wid * NB + c * CB
    ci, _, _, si = bufs[s]
    return (
        pltpu.make_async_copy(ctx_hbm.at[pl.ds(rb * W, CB * W)], ci, si),
    )

  def g_copies(s):
    ci, cr, sg, _ = bufs[s]
    return (pltpu.make_async_copy(itab_hbm.at[ci], cr, sg),)

  def compute(s, c):
    ctx_rows_v = bufs[s][1]

    def elem_body(b, carry2):
      for v in range(D // 16):
        rows = [ctx_rows_v[b * W + w, pl.ds(v * 16, 16)] for w in range(W)]
        sums_v[b, pl.ds(v * 16, 16)] = _tree_sum(rows)
      return carry2

    lax.fori_loop(0, CB, elem_body, 0)
    rb = wid * NB + c * CB
    pltpu.sync_copy(sums_v, out_hbm.at[pl.ds(rb, CB)])

  _pipeline(i_copies, g_copies, compute)


def _sc_dots_kernel(tgt_hbm, neg_hbm, otab_hbm, sums_hbm, out_hbm,
                    tgt_idx_a, neg_idx_a, tgt_rows_a, neg_rows_a, sums_a,
                    tgt_idx_b, neg_idx_b, tgt_rows_b, neg_rows_b, sums_b,
                    part0_v, part1_v, logits_v,
                    sem_ga, sem_gb, sem_ia, sem_ib):
  wid = lax.axis_index("s") * NC + lax.axis_index("c")
  lane = lax.iota(jnp.int32, 16)
  bufs = (
      (tgt_idx_a, neg_idx_a, tgt_rows_a, neg_rows_a, sums_a, sem_ga, sem_ia),
      (tgt_idx_b, neg_idx_b, tgt_rows_b, neg_rows_b, sums_b, sem_gb, sem_ib),
  )

  def i_copies(c, s):
    rb = wid * NB + c * CB
    ti, ni, _, _, sm, _, si = bufs[s]
    return (
        pltpu.make_async_copy(tgt_hbm.at[pl.ds(rb, CB)], ti, si),
        pltpu.make_async_copy(neg_hbm.at[pl.ds(rb * K, CB * K)], ni, si),
        pltpu.make_async_copy(sums_hbm.at[pl.ds(rb, CB)], sm, si),
    )

  def g_copies(s):
    ti, ni, tr, nr, _, sg, _ = bufs[s]
    return (
        pltpu.make_async_copy(otab_hbm.at[ti], tr, sg),
        pltpu.make_async_copy(otab_hbm.at[ni], nr, sg),
    )

  def compute(s, c):
    tgt_rows_v, neg_rows_v, sums_v = bufs[s][2], bufs[s][3], bufs[s][4]

    def scatter_dots(b, part_v):
      mean = [sums_v[b, pl.ds(v * 16, 16)] for v in range(D // 16)]

      def dot_partial(rows_ref, r, k):
        prods = [mean[v] * rows_ref[r, pl.ds(v * 16, 16)]
                 for v in range(D // 16)]
        plsc.store_scatter(part_v, [lane * PAD + k], _tree_sum(prods))

      dot_partial(tgt_rows_v, b, 0)
      for k in range(K):
        dot_partial(neg_rows_v, b * K + k, k + 1)

    def reduce_part(b, part_v):
      acc1 = _tree_sum([part_v[pl.ds(i * PAD, 16)] for i in range(16)])
      acc2 = _tree_sum([part_v[pl.ds(i * PAD + 16, 16)] for i in range(16)])
      logits_v[b, pl.ds(0, 16)] = acc1
      logits_v[b, pl.ds(16, 16)] = acc2

    def pair_body(h, carry2):
      b0 = 2 * h
      scatter_dots(b0, part0_v)
      scatter_dots(b0 + 1, part1_v)
      reduce_part(b0, part0_v)
      reduce_part(b0 + 1, part1_v)
      return carry2

    lax.fori_loop(0, CB // 2, pair_body, 0)
    rb = wid * NB + c * CB
    pltpu.sync_copy(logits_v, out_hbm.at[pl.ds(rb, CB)])

  _pipeline(i_copies, g_copies, compute)


_SC_PARAMS = pltpu.CompilerParams(
    needs_layout_passes=False, use_tc_tiling_on_sc=False)


@jax.jit
def _sc_loss_parts(ctx_flat, tgt_flat, neg_flat, itab, otab):
  mesh = plsc.VectorSubcoreMesh(core_axis_name="c", subcore_axis_name="s")
  sums = pl.kernel(
      _sc_sums_kernel,
      mesh=mesh,
      out_type=jax.ShapeDtypeStruct((B, D), jnp.float32),
      compiler_params=_SC_PARAMS,
      scratch_types=[
          pltpu.VMEM((CB * W,), jnp.int32),
          pltpu.VMEM((CB * W, D), jnp.float32),
          pltpu.VMEM((CB * W,), jnp.int32),
          pltpu.VMEM((CB * W, D), jnp.float32),
          pltpu.VMEM((CB, D), jnp.float32),
          pltpu.SemaphoreType.DMA,
          pltpu.SemaphoreType.DMA,
          pltpu.SemaphoreType.DMA,
          pltpu.SemaphoreType.DMA,
      ],
  )(ctx_flat, itab)
  logits = pl.kernel(
      _sc_dots_kernel,
      mesh=mesh,
      out_type=jax.ShapeDtypeStruct((B, PAD), jnp.float32),
      compiler_params=_SC_PARAMS,
      scratch_types=[
          pltpu.VMEM((CB,), jnp.int32),
          pltpu.VMEM((CB * K,), jnp.int32),
          pltpu.VMEM((CB, D), jnp.float32),
          pltpu.VMEM((CB * K, D), jnp.float32),
          pltpu.VMEM((CB, D), jnp.float32),
          pltpu.VMEM((CB,), jnp.int32),
          pltpu.VMEM((CB * K,), jnp.int32),
          pltpu.VMEM((CB, D), jnp.float32),
          pltpu.VMEM((CB * K, D), jnp.float32),
          pltpu.VMEM((CB, D), jnp.float32),
          pltpu.VMEM((16 * PAD,), jnp.float32),
          pltpu.VMEM((16 * PAD,), jnp.float32),
          pltpu.VMEM((CB, PAD), jnp.float32),
          pltpu.SemaphoreType.DMA,
          pltpu.SemaphoreType.DMA,
          pltpu.SemaphoreType.DMA,
          pltpu.SemaphoreType.DMA,
      ],
  )(tgt_flat, neg_flat, otab, sums)
  return logits


def _bce_body(l_ref, o_ref):
  x = l_ref[...] * jnp.float32(1.0 / W)
  col = lax.broadcasted_iota(jnp.int32, x.shape, 1)
  label = (col == 0).astype(x.dtype)
  loss = jnp.maximum(x, 0.0) - x * label + jnp.log1p(jnp.exp(-jnp.abs(x)))
  loss = jnp.where(col < (K + 1), loss, 0.0)
  o_ref[0, 0] = jnp.sum(loss) / jnp.float32(B * (K + 1))


@jax.jit
def _bce_mean(logits):
  out = pl.pallas_call(
      _bce_body,
      out_shape=jax.ShapeDtypeStruct((1, 1), jnp.float32),
      in_specs=[pl.BlockSpec(memory_space=pltpu.VMEM)],
      out_specs=pl.BlockSpec(memory_space=pltpu.SMEM),
  )(logits)
  return out[0, 0]


def kernel(context, target, negatives, input_table, output_table):
  ctx_flat = context.astype(jnp.int32).reshape(-1)
  tgt_flat = target.astype(jnp.int32).reshape(-1)
  neg_flat = negatives.astype(jnp.int32).reshape(-1)
  logits = _sc_loss_parts(ctx_flat, tgt_flat, neg_flat,
                          input_table, output_table)
  return _bce_mean(logits)
